# baseline (speedup 1.0000x reference)
import functools

import jax
import jax.numpy as jnp
from jax import lax
from jax.experimental import pallas as pl
from jax.experimental.pallas import tpu as pltpu
from jax.experimental.pallas import tpu_sc as plsc

NUM_CORES = 2
NUM_SUBCORES = 16
NUM_WORKERS = NUM_CORES * NUM_SUBCORES
NBUF = 8
SLACK = 2


def _make_emb_kernel(batch: int, seq: int, vocab: int, d: int):
  per_w = batch // NUM_WORKERS
  assert (per_w - NBUF) % NBUF == 0 and per_w > NBUF + SLACK
  n_groups = (per_w - NBUF) // NBUF
  mesh = plsc.VectorSubcoreMesh(core_axis_name="c", subcore_axis_name="s")

  @functools.partial(
      pl.kernel,
      mesh=mesh,
      out_type=jax.ShapeDtypeStruct((batch, seq, d), jnp.float32),
      scratch_types=[
          pltpu.VMEM((per_w, seq), jnp.int32),
          pltpu.VMEM((NBUF, seq, d), jnp.float32),
      ] + [pltpu.SemaphoreType.DMA] * (2 * NBUF),
  )
  def emb(idx_hbm, tab_hbm, out_hbm, idx_v, rows_v, *sems):
    gsems, ssems = sems[:NBUF], sems[NBUF:]
    wid = lax.axis_index("s") * NUM_CORES + lax.axis_index("c")
    base = wid * per_w
    pltpu.sync_copy(idx_hbm.at[wid], idx_v)

    def gather_start(k, b):
      pltpu.async_copy(tab_hbm.at[idx_v.at[k]], rows_v.at[b], gsems[b])

    def gather_wait(k, b):
      pltpu.make_async_copy(
          tab_hbm.at[idx_v.at[k]], rows_v.at[b], gsems[b]).wait()

    def store_start(k, b):
      pltpu.async_copy(rows_v.at[b], out_hbm.at[base + k], ssems[b])

    def store_wait(k, b):
      pltpu.make_async_copy(
          rows_v.at[b], out_hbm.at[base + k], ssems[b]).wait()

    for b in range(NBUF):
      gather_start(b, b)
    for k in range(SLACK):
      gather_wait(k, k)
      store_start(k, k)

    def group(g):
      for i in range(NBUF):
        k = SLACK + g * NBUF + i
        b = (SLACK + i) % NBUF
        br = i
        store_wait(k - SLACK, br)
        gather_start(k - SLACK + NBUF, br)
        gather_wait(k, b)
        store_start(k, b)

    pl.loop(0, n_groups)(group)

    for k in range(per_w - NBUF + SLACK, per_w):
      store_wait(k - SLACK, (k - SLACK) % NBUF)
      gather_wait(k, k % NBUF)
      store_start(k, k % NBUF)
    for k in range(per_w - SLACK, per_w):
      store_wait(k, k % NBUF)

  return emb


N_CALLS = 2


def kernel(word_indices, embedding_table):
  batch, seq = word_indices.shape
  vocab, d = embedding_table.shape
  sub = batch // N_CALLS
  emb = _make_emb_kernel(sub, seq, vocab, d)
  outs = []
  for i in range(N_CALLS):
    idx3 = word_indices[i * sub:(i + 1) * sub].astype(jnp.int32).reshape(
        NUM_WORKERS, sub // NUM_WORKERS, seq)
    outs.append(emb(idx3, embedding_table))
  return jnp.concatenate(outs, axis=0)


# speedup vs baseline: 2.8640x; 2.8640x over previous
import functools

import jax
import jax.numpy as jnp
from jax import lax
from jax.experimental import pallas as pl
from jax.experimental.pallas import tpu as pltpu
from jax.experimental.pallas import tpu_sc as plsc

NUM_CORES = 2
NUM_SUBCORES = 16
NUM_WORKERS = NUM_CORES * NUM_SUBCORES
NBUF = 5
SLACK = 2


def _make_emb_kernel(batch: int, seq: int, vocab: int, d: int):
  per_w = batch // NUM_WORKERS
  assert (seq - NBUF) % NBUF == 0 and seq > NBUF + SLACK
  n_groups = (seq - NBUF) // NBUF
  mesh = plsc.VectorSubcoreMesh(core_axis_name="c", subcore_axis_name="s")

  @functools.partial(
      pl.kernel,
      mesh=mesh,
      out_type=jax.ShapeDtypeStruct((seq, batch, d), jnp.float32),
      scratch_types=[
          pltpu.VMEM((seq, per_w), jnp.int32),
          pltpu.VMEM((NBUF, per_w, d), jnp.float32),
      ] + [pltpu.SemaphoreType.DMA] * (2 * NBUF),
  )
  def emb(idx_hbm, tab_hbm, out_hbm, idx_v, rows_v, *sems):
    gsems, ssems = sems[:NBUF], sems[NBUF:]
    wid = lax.axis_index("s") * NUM_CORES + lax.axis_index("c")
    base = wid * per_w
    pltpu.sync_copy(idx_hbm.at[wid], idx_v)

    def gather_start(l, b):
      pltpu.async_copy(tab_hbm.at[idx_v.at[l]], rows_v.at[b], gsems[b])

    def gather_wait(l, b):
      pltpu.make_async_copy(
          tab_hbm.at[idx_v.at[l]], rows_v.at[b], gsems[b]).wait()

    def store_start(l, b):
      pltpu.async_copy(
          rows_v.at[b], out_hbm.at[l, pl.ds(base, per_w)], ssems[b])

    def store_wait(l, b):
      pltpu.make_async_copy(
          rows_v.at[b], out_hbm.at[l, pl.ds(base, per_w)], ssems[b]).wait()

    for b in range(NBUF):
      gather_start(b, b)
    for l in range(SLACK):
      gather_wait(l, l)
      store_start(l, l)

    def group(g):
      for i in range(NBUF):
        l = SLACK + g * NBUF + i
        b = (SLACK + i) % NBUF
        br = i
        store_wait(l - SLACK, br)
        gather_start(l - SLACK + NBUF, br)
        gather_wait(l, b)
        store_start(l, b)

    pl.loop(0, n_groups)(group)

    for l in range(seq - NBUF + SLACK, seq):
      store_wait(l - SLACK, (l - SLACK) % NBUF)
      gather_wait(l, l % NBUF)
      store_start(l, l % NBUF)
    for l in range(seq - SLACK, seq):
      store_wait(l, l % NBUF)

  return emb


def kernel(word_indices, embedding_table):
  batch, seq = word_indices.shape
  vocab, d = embedding_table.shape
  per_w = batch // NUM_WORKERS
  idx3 = word_indices.astype(jnp.int32).reshape(
      NUM_WORKERS, per_w, seq).transpose(0, 2, 1)
  emb = _make_emb_kernel(batch, seq, vocab, d)
  out_t = emb(idx3, embedding_table)
  return jnp.transpose(out_t, (1, 0, 2))


# speedup vs baseline: 2.8976x; 1.0117x over previous
import functools

import jax
import jax.numpy as jnp
from jax import lax
from jax.experimental import pallas as pl
from jax.experimental.pallas import tpu as pltpu
from jax.experimental.pallas import tpu_sc as plsc

NUM_CORES = 2
NUM_SUBCORES = 16
NUM_WORKERS = NUM_CORES * NUM_SUBCORES
NBUF = 10
SLACK = 3
HALVES = 2


def _make_emb_kernel(batch: int, seq: int, vocab: int, d: int):
  per_w = batch // NUM_WORKERS
  half = per_w // HALVES
  n_steps = seq * HALVES
  assert (n_steps - NBUF) % NBUF == 0 and n_steps > NBUF + SLACK
  n_groups = (n_steps - NBUF) // NBUF
  mesh = plsc.VectorSubcoreMesh(core_axis_name="c", subcore_axis_name="s")

  @functools.partial(
      pl.kernel,
      mesh=mesh,
      out_type=jax.ShapeDtypeStruct((seq, batch, d), jnp.float32),
      scratch_types=[
          pltpu.VMEM((seq, per_w), jnp.int32),
          pltpu.VMEM((NBUF, half, d), jnp.float32),
      ] + [pltpu.SemaphoreType.DMA] * (2 * NBUF),
  )
  def emb(idx_hbm, tab_hbm, out_hbm, idx_v, rows_v, *sems):
    gsems, ssems = sems[:NBUF], sems[NBUF:]
    wid = lax.axis_index("s") * NUM_CORES + lax.axis_index("c")
    base = wid * per_w
    pltpu.sync_copy(idx_hbm.at[wid], idx_v)

    def gather_start(step, b):
      l, h = step // HALVES, step % HALVES
      pltpu.async_copy(
          tab_hbm.at[idx_v.at[l, pl.ds(h * half, half)]], rows_v.at[b],
          gsems[b])

    def gather_wait(step, b):
      l, h = step // HALVES, step % HALVES
      pltpu.make_async_copy(
          tab_hbm.at[idx_v.at[l, pl.ds(h * half, half)]], rows_v.at[b],
          gsems[b]).wait()

    def store_start(step, b):
      l, h = step // HALVES, step % HALVES
      pltpu.async_copy(
          rows_v.at[b], out_hbm.at[l, pl.ds(base + h * half, half)], ssems[b])

    def store_wait(step, b):
      l, h = step // HALVES, step % HALVES
      pltpu.make_async_copy(
          rows_v.at[b], out_hbm.at[l, pl.ds(base + h * half, half)],
          ssems[b]).wait()

    for b in range(NBUF):
      gather_start(b, b)
    for s in range(SLACK):
      gather_wait(s, s)
      store_start(s, s)

    def group(g):
      for i in range(NBUF):
        s = SLACK + g * NBUF + i
        b = (SLACK + i) % NBUF
        br = i
        store_wait(s - SLACK, br)
        gather_start(s - SLACK + NBUF, br)
        gather_wait(s, b)
        store_start(s, b)

    pl.loop(0, n_groups)(group)

    for s in range(n_steps - NBUF + SLACK, n_steps):
      store_wait(s - SLACK, (s - SLACK) % NBUF)
      gather_wait(s, s % NBUF)
      store_start(s, s % NBUF)
    for s in range(n_steps - SLACK, n_steps):
      store_wait(s, s % NBUF)

  return emb


def kernel(word_indices, embedding_table):
  batch, seq = word_indices.shape
  vocab, d = embedding_table.shape
  per_w = batch // NUM_WORKERS
  idx3 = word_indices.astype(jnp.int32).reshape(
      NUM_WORKERS, per_w, seq).transpose(0, 2, 1)
  emb = _make_emb_kernel(batch, seq, vocab, d)
  out_t = emb(idx3, embedding_table)
  return jnp.transpose(out_t, (1, 0, 2))
